# baseline (speedup 1.0000x reference)
import jax
import jax.numpy as jnp
from jax.experimental import pallas as pl
from jax.experimental.pallas import tpu as pltpu

_HC = 4


def _balanced_add(ts):
    n = len(ts)
    if n == 1:
        return ts[0]
    return _balanced_add(ts[: n // 2]) + _balanced_add(ts[n // 2:])


def _pairs(cin):
    return [(i, j) for i in range(cin) for j in range(i, cin)]


def _dw_accs(slab_ref, top_ref, bot_ref, dw_ref, xp_ref):
    i = pl.program_id(0)
    ng = pl.num_programs(0)
    cin, hc, w, _ = slab_ref.shape

    xp_ref[:, :, 0:1, :] = jnp.zeros_like(xp_ref[:, :, 0:1, :])
    xp_ref[:, :, w + 1:w + 2, :] = jnp.zeros_like(xp_ref[:, :, w + 1:w + 2, :])
    xp_ref[:, 1:hc + 1, 1:w + 1, :] = jnp.maximum(slab_ref[...], 0.0)

    @pl.when(i == 0)
    def _():
        xp_ref[:, 0:1, 1:w + 1, :] = jnp.zeros_like(top_ref[...])

    @pl.when(i > 0)
    def _():
        xp_ref[:, 0:1, 1:w + 1, :] = jnp.maximum(top_ref[...], 0.0)

    @pl.when(i == ng - 1)
    def _():
        xp_ref[:, hc + 1:hc + 2, 1:w + 1, :] = jnp.zeros_like(bot_ref[...])

    @pl.when(i < ng - 1)
    def _():
        xp_ref[:, hc + 1:hc + 2, 1:w + 1, :] = jnp.maximum(bot_ref[...], 0.0)

    accs = []
    for ci in range(cin):
        taps = [xp_ref[ci, kh:kh + hc, kw:kw + w, :] * dw_ref[ci, kh * 3 + kw]
                for kh in range(3) for kw in range(3)]
        accs.append(_balanced_add(taps))
    return accs


def _moments_kernel(slab_ref, top_ref, bot_ref, dw_ref, mom_ref, xp_ref):
    cin, hc, w, nn = slab_ref.shape
    accs = [jnp.reshape(a, (hc * w, nn))
            for a in _dw_accs(slab_ref, top_ref, bot_ref, dw_ref, xp_ref)]
    for ci in range(cin):
        mom_ref[0, ci] = jnp.sum(accs[ci], axis=0)
    for k, (ci, cj) in enumerate(_pairs(cin)):
        mom_ref[0, cin + k] = jnp.sum(accs[ci] * accs[cj], axis=0)


def _bn_apply_kernel(slab_ref, top_ref, bot_ref, dw_ref, pm_ref, shift_ref,
                     o_ref, xp_ref):
    cin = slab_ref.shape[0]
    cout = pm_ref.shape[0]
    accs = _dw_accs(slab_ref, top_ref, bot_ref, dw_ref, xp_ref)
    for co in range(cout):
        y = _balanced_add([accs[ci] * pm_ref[co, ci] for ci in range(cin)])
        o_ref[co] = y + shift_ref[co, 0]


def kernel(x_nchw, dw_w, pw_w, gamma, beta):
    n, cin, h, w = x_nchw.shape
    cout = pw_w.shape[0]
    hc = _HC
    assert h % hc == 0
    grid = (h // hc,)
    eps = 1e-5

    xt = jnp.transpose(x_nchw.astype(jnp.float32), (1, 2, 3, 0))

    dw = dw_w.astype(jnp.float32).reshape(cin, 9)
    pmat = pw_w.astype(jnp.float32).reshape(cout, cin)

    cparams = pltpu.CompilerParams(dimension_semantics=("parallel",),
                                   vmem_limit_bytes=64 * 1024 * 1024)
    smem = pl.BlockSpec(memory_space=pltpu.SMEM)
    slab_spec = pl.BlockSpec((cin, hc, w, n), lambda i: (0, i, 0, 0))
    top_spec = pl.BlockSpec((cin, 1, w, n),
                            lambda i: (0, jnp.maximum(i * hc - 1, 0), 0, 0))
    bot_spec = pl.BlockSpec((cin, 1, w, n),
                            lambda i: (0, jnp.minimum(i * hc + hc, h - 1), 0, 0))

    nmom = cin + len(_pairs(cin))
    moments = pl.pallas_call(
        _moments_kernel,
        out_shape=jax.ShapeDtypeStruct((h // hc, nmom, n), jnp.float32),
        grid=grid,
        in_specs=[slab_spec, top_spec, bot_spec, smem],
        out_specs=pl.BlockSpec((1, nmom, n), lambda i: (i, 0, 0)),
        scratch_shapes=[pltpu.VMEM((cin, hc + 2, w + 2, n), jnp.float32)],
        compiler_params=cparams,
    )(xt, xt, xt, dw)

    mom = jnp.sum(moments, axis=(0, 2))
    s = mom[:cin]
    gram = jnp.zeros((cin, cin), jnp.float32)
    for k, (ci, cj) in enumerate(_pairs(cin)):
        v = mom[cin + k]
        gram = gram.at[ci, cj].set(v)
        if ci != cj:
            gram = gram.at[cj, ci].set(v)
    count = jnp.float32(n * h * w)
    mean = (pmat @ s).reshape(cout, 1) / count
    ey2 = jnp.sum((pmat @ gram) * pmat, axis=1).reshape(cout, 1) / count
    var = ey2 - mean * mean
    inv = jax.lax.rsqrt(var + eps)
    scale = gamma.astype(jnp.float32).reshape(cout, 1) * inv
    shift = beta.astype(jnp.float32).reshape(cout, 1) - mean * scale
    pmat_s = pmat * scale

    yt = pl.pallas_call(
        _bn_apply_kernel,
        out_shape=jax.ShapeDtypeStruct((cout, h, w, n), jnp.float32),
        grid=grid,
        in_specs=[slab_spec, top_spec, bot_spec, smem, smem, smem],
        out_specs=pl.BlockSpec((cout, hc, w, n), lambda i: (0, i, 0, 0)),
        scratch_shapes=[pltpu.VMEM((cin, hc + 2, w + 2, n), jnp.float32)],
        compiler_params=cparams,
    )(xt, xt, xt, dw, pmat_s, shift)

    return jnp.transpose(yt, (3, 0, 1, 2))


# speedup vs baseline: 1.7640x; 1.7640x over previous
import jax
import jax.numpy as jnp
from jax.experimental import pallas as pl
from jax.experimental.pallas import tpu as pltpu

_NL = 128


def _balanced_add(ts):
    n = len(ts)
    if n == 1:
        return ts[0]
    return _balanced_add(ts[: n // 2]) + _balanced_add(ts[n // 2:])


def _pairs(cin):
    return [(i, j) for i in range(cin) for j in range(i, cin)]


def _dw_accs(x_ref, dw_ref, xp_ref):
    cin, h, w, _ = x_ref.shape
    xp_ref[:, :, 0:1, :] = jnp.zeros_like(xp_ref[:, :, 0:1, :])
    xp_ref[:, :, w + 1:w + 2, :] = jnp.zeros_like(xp_ref[:, :, w + 1:w + 2, :])
    xp_ref[:, 0:1, :, :] = jnp.zeros_like(xp_ref[:, 0:1, :, :])
    xp_ref[:, h + 1:h + 2, :, :] = jnp.zeros_like(xp_ref[:, h + 1:h + 2, :, :])
    xp_ref[:, 1:h + 1, 1:w + 1, :] = jnp.maximum(x_ref[...], 0.0)

    accs = []
    for ci in range(cin):
        taps = [xp_ref[ci, kh:kh + h, kw:kw + w, :] * dw_ref[ci, kh * 3 + kw]
                for kh in range(3) for kw in range(3)]
        accs.append(_balanced_add(taps))
    return accs


def _moments_kernel(x_ref, dw_ref, mom_ref, xp_ref):
    cin, h, w, nl = x_ref.shape
    accs = [jnp.reshape(a, (h * w, nl))
            for a in _dw_accs(x_ref, dw_ref, xp_ref)]
    for ci in range(cin):
        mom_ref[0, ci] = jnp.sum(accs[ci], axis=0)
    for k, (ci, cj) in enumerate(_pairs(cin)):
        mom_ref[0, cin + k] = jnp.sum(accs[ci] * accs[cj], axis=0)


def _bn_apply_kernel(x_ref, dw_ref, pm_ref, shift_ref, o_ref, xp_ref):
    cin = x_ref.shape[0]
    cout = pm_ref.shape[0]
    accs = _dw_accs(x_ref, dw_ref, xp_ref)
    for co in range(cout):
        y = _balanced_add([accs[ci] * pm_ref[co, ci] for ci in range(cin)])
        o_ref[co] = y + shift_ref[co, 0]


def kernel(x_nchw, dw_w, pw_w, gamma, beta):
    n, cin, h, w = x_nchw.shape
    cout = pw_w.shape[0]
    nl = _NL
    assert n % nl == 0
    grid = (n // nl,)
    eps = 1e-5

    xt = jnp.transpose(x_nchw.astype(jnp.float32), (1, 2, 3, 0))

    dw = dw_w.astype(jnp.float32).reshape(cin, 9)
    pmat = pw_w.astype(jnp.float32).reshape(cout, cin)

    cparams = pltpu.CompilerParams(dimension_semantics=("parallel",),
                                   vmem_limit_bytes=64 * 1024 * 1024)
    smem = pl.BlockSpec(memory_space=pltpu.SMEM)

    nmom = cin + len(_pairs(cin))
    moments = pl.pallas_call(
        _moments_kernel,
        out_shape=jax.ShapeDtypeStruct((n // nl, nmom, nl), jnp.float32),
        grid=grid,
        in_specs=[pl.BlockSpec((cin, h, w, nl), lambda i: (0, 0, 0, i)),
                  smem],
        out_specs=pl.BlockSpec((1, nmom, nl), lambda i: (i, 0, 0)),
        scratch_shapes=[pltpu.VMEM((cin, h + 2, w + 2, nl), jnp.float32)],
        compiler_params=cparams,
    )(xt, dw)

    mom = jnp.sum(moments, axis=(0, 2))
    s = mom[:cin]
    gram = jnp.zeros((cin, cin), jnp.float32)
    for k, (ci, cj) in enumerate(_pairs(cin)):
        v = mom[cin + k]
        gram = gram.at[ci, cj].set(v)
        if ci != cj:
            gram = gram.at[cj, ci].set(v)
    count = jnp.float32(n * h * w)
    mean = (pmat @ s).reshape(cout, 1) / count
    ey2 = jnp.sum((pmat @ gram) * pmat, axis=1).reshape(cout, 1) / count
    var = ey2 - mean * mean
    inv = jax.lax.rsqrt(var + eps)
    scale = gamma.astype(jnp.float32).reshape(cout, 1) * inv
    shift = beta.astype(jnp.float32).reshape(cout, 1) - mean * scale
    pmat_s = pmat * scale

    yt = pl.pallas_call(
        _bn_apply_kernel,
        out_shape=jax.ShapeDtypeStruct((cout, h, w, n), jnp.float32),
        grid=grid,
        in_specs=[pl.BlockSpec((cin, h, w, nl), lambda i: (0, 0, 0, i)),
                  smem, smem, smem],
        out_specs=pl.BlockSpec((cout, h, w, nl), lambda i: (0, 0, 0, i)),
        scratch_shapes=[pltpu.VMEM((cin, h + 2, w + 2, nl), jnp.float32)],
        compiler_params=cparams,
    )(xt, dw, pmat_s, shift)

    return jnp.transpose(yt, (3, 0, 1, 2))


# speedup vs baseline: 1.7654x; 1.0008x over previous
import jax
import jax.numpy as jnp
from jax.experimental import pallas as pl
from jax.experimental.pallas import tpu as pltpu

_NL = 128


def _balanced_add(ts):
    n = len(ts)
    if n == 1:
        return ts[0]
    return _balanced_add(ts[: n // 2]) + _balanced_add(ts[n // 2:])


def _pairs(cin):
    return [(i, j) for i in range(cin) for j in range(i, cin)]


def _dw_accs(x_ref, dw_ref, xp_ref):
    cin, h, w, _ = x_ref.shape
    xp_ref[:, :, 0:1, :] = jnp.zeros_like(xp_ref[:, :, 0:1, :])
    xp_ref[:, :, w + 1:w + 2, :] = jnp.zeros_like(xp_ref[:, :, w + 1:w + 2, :])
    xp_ref[:, 0:1, :, :] = jnp.zeros_like(xp_ref[:, 0:1, :, :])
    xp_ref[:, h + 1:h + 2, :, :] = jnp.zeros_like(xp_ref[:, h + 1:h + 2, :, :])
    xp_ref[:, 1:h + 1, 1:w + 1, :] = jnp.maximum(x_ref[...], 0.0)

    accs = []
    for ci in range(cin):
        taps = [xp_ref[ci, kh:kh + h, kw:kw + w, :] * dw_ref[ci, kh * 3 + kw]
                for kh in range(3) for kw in range(3)]
        accs.append(_balanced_add(taps))
    return accs


def _moments_kernel(x_ref, dw_ref, mom_ref, xp_ref):
    cin = x_ref.shape[0]
    accs = _dw_accs(x_ref, dw_ref, xp_ref)
    for ci in range(cin):
        mom_ref[0, ci] = jnp.sum(accs[ci], axis=(0, 1))
    for k, (ci, cj) in enumerate(_pairs(cin)):
        mom_ref[0, cin + k] = jnp.sum(accs[ci] * accs[cj], axis=(0, 1))


def _bn_apply_kernel(x_ref, dw_ref, pm_ref, shift_ref, o_ref, xp_ref):
    cin = x_ref.shape[0]
    cout = pm_ref.shape[0]
    accs = _dw_accs(x_ref, dw_ref, xp_ref)
    for co in range(cout):
        y = _balanced_add([accs[ci] * pm_ref[co, ci] for ci in range(cin)])
        o_ref[co] = y + shift_ref[co, 0]


def kernel(x_nchw, dw_w, pw_w, gamma, beta):
    n, cin, h, w = x_nchw.shape
    cout = pw_w.shape[0]
    nl = _NL
    assert n % nl == 0
    grid = (n // nl,)
    eps = 1e-5

    xt = jnp.transpose(x_nchw.astype(jnp.float32), (1, 2, 3, 0))

    dw = dw_w.astype(jnp.float32).reshape(cin, 9)
    pmat = pw_w.astype(jnp.float32).reshape(cout, cin)

    cparams = pltpu.CompilerParams(dimension_semantics=("parallel",),
                                   vmem_limit_bytes=64 * 1024 * 1024)
    smem = pl.BlockSpec(memory_space=pltpu.SMEM)

    nmom = cin + len(_pairs(cin))
    moments = pl.pallas_call(
        _moments_kernel,
        out_shape=jax.ShapeDtypeStruct((n // nl, nmom, nl), jnp.float32),
        grid=grid,
        in_specs=[pl.BlockSpec((cin, h, w, nl), lambda i: (0, 0, 0, i)),
                  smem],
        out_specs=pl.BlockSpec((1, nmom, nl), lambda i: (i, 0, 0)),
        scratch_shapes=[pltpu.VMEM((cin, h + 2, w + 2, nl), jnp.float32)],
        compiler_params=cparams,
    )(xt, dw)

    mom = jnp.sum(moments, axis=(0, 2))
    s = mom[:cin]
    gram = jnp.zeros((cin, cin), jnp.float32)
    for k, (ci, cj) in enumerate(_pairs(cin)):
        v = mom[cin + k]
        gram = gram.at[ci, cj].set(v)
        if ci != cj:
            gram = gram.at[cj, ci].set(v)
    count = jnp.float32(n * h * w)
    mean = (pmat @ s).reshape(cout, 1) / count
    ey2 = jnp.sum((pmat @ gram) * pmat, axis=1).reshape(cout, 1) / count
    var = ey2 - mean * mean
    inv = jax.lax.rsqrt(var + eps)
    scale = gamma.astype(jnp.float32).reshape(cout, 1) * inv
    shift = beta.astype(jnp.float32).reshape(cout, 1) - mean * scale
    pmat_s = pmat * scale

    yt = pl.pallas_call(
        _bn_apply_kernel,
        out_shape=jax.ShapeDtypeStruct((cout, h, w, n), jnp.float32),
        grid=grid,
        in_specs=[pl.BlockSpec((cin, h, w, nl), lambda i: (0, 0, 0, i)),
                  smem, smem, smem],
        out_specs=pl.BlockSpec((cout, h, w, nl), lambda i: (0, 0, 0, i)),
        scratch_shapes=[pltpu.VMEM((cin, h + 2, w + 2, nl), jnp.float32)],
        compiler_params=cparams,
    )(xt, dw, pmat_s, shift)

    return jnp.transpose(yt, (3, 0, 1, 2))


# speedup vs baseline: 1.9265x; 1.0913x over previous
import jax
import jax.numpy as jnp
from jax.experimental import pallas as pl
from jax.experimental.pallas import tpu as pltpu

_NL = 128


def _balanced_add(ts):
    n = len(ts)
    if n == 1:
        return ts[0]
    return _balanced_add(ts[: n // 2]) + _balanced_add(ts[n // 2:])


def _pairs(cin):
    return [(i, j) for i in range(cin) for j in range(i, cin)]


def _dw_accs(x_ref, dw_ref, xp_ref):
    cin, h, w, _ = x_ref.shape
    xp_ref[:, :, 0:1, :] = jnp.zeros_like(xp_ref[:, :, 0:1, :])
    xp_ref[:, :, w + 1:w + 2, :] = jnp.zeros_like(xp_ref[:, :, w + 1:w + 2, :])
    xp_ref[:, 0:1, :, :] = jnp.zeros_like(xp_ref[:, 0:1, :, :])
    xp_ref[:, h + 1:h + 2, :, :] = jnp.zeros_like(xp_ref[:, h + 1:h + 2, :, :])
    xp_ref[:, 1:h + 1, 1:w + 1, :] = jnp.maximum(x_ref[...], 0.0)

    accs = []
    for ci in range(cin):
        taps = [xp_ref[ci, kh:kh + h, kw:kw + w, :] * dw_ref[ci, kh * 3 + kw]
                for kh in range(3) for kw in range(3)]
        accs.append(_balanced_add(taps))
    return accs


def _conv_ys(x_ref, dw_ref, pm_ref, xp_ref):
    cin = x_ref.shape[0]
    cout = pm_ref.shape[0]
    accs = _dw_accs(x_ref, dw_ref, xp_ref)
    ys = [None] * cout
    for ci in range(cin):
        for co in range(cout):
            t = accs[ci] * pm_ref[co, ci]
            ys[co] = t if ci == 0 else ys[co] + t
    return ys


def _moments_kernel(x_ref, dw_ref, pm_ref, mom_ref, xp_ref):
    ys = _conv_ys(x_ref, dw_ref, pm_ref, xp_ref)
    for co, y in enumerate(ys):
        mom_ref[0, co, 0] = jnp.sum(y, axis=(0, 1))
        mom_ref[0, co, 1] = jnp.sum(y * y, axis=(0, 1))


def _bn_apply_kernel(x_ref, dw_ref, pm_ref, shift_ref, o_ref, xp_ref):
    ys = _conv_ys(x_ref, dw_ref, pm_ref, xp_ref)
    for co, y in enumerate(ys):
        o_ref[co] = y + shift_ref[co, 0]


def kernel(x_nchw, dw_w, pw_w, gamma, beta):
    n, cin, h, w = x_nchw.shape
    cout = pw_w.shape[0]
    nl = _NL
    assert n % nl == 0
    grid = (n // nl,)
    eps = 1e-5

    xt = jnp.transpose(x_nchw.astype(jnp.float32), (1, 2, 3, 0))

    dw = dw_w.astype(jnp.float32).reshape(cin, 9)
    pmat = pw_w.astype(jnp.float32).reshape(cout, cin)

    cparams = pltpu.CompilerParams(dimension_semantics=("parallel",),
                                   vmem_limit_bytes=64 * 1024 * 1024)
    smem = pl.BlockSpec(memory_space=pltpu.SMEM)

    moments = pl.pallas_call(
        _moments_kernel,
        out_shape=jax.ShapeDtypeStruct((n // nl, cout, 2, nl), jnp.float32),
        grid=grid,
        in_specs=[pl.BlockSpec((cin, h, w, nl), lambda i: (0, 0, 0, i)),
                  smem, smem],
        out_specs=pl.BlockSpec((1, cout, 2, nl), lambda i: (i, 0, 0, 0)),
        scratch_shapes=[pltpu.VMEM((cin, h + 2, w + 2, nl), jnp.float32)],
        compiler_params=cparams,
    )(xt, dw, pmat)

    tot = jnp.sum(moments, axis=(0, 3))
    count = jnp.float32(n * h * w)
    mean = tot[:, 0:1] / count
    var = tot[:, 1:2] / count - mean * mean
    inv = jax.lax.rsqrt(var + eps)
    scale = gamma.astype(jnp.float32).reshape(cout, 1) * inv
    shift = beta.astype(jnp.float32).reshape(cout, 1) - mean * scale
    pmat_s = pmat * scale

    yt = pl.pallas_call(
        _bn_apply_kernel,
        out_shape=jax.ShapeDtypeStruct((cout, h, w, n), jnp.float32),
        grid=grid,
        in_specs=[pl.BlockSpec((cin, h, w, nl), lambda i: (0, 0, 0, i)),
                  smem, smem, smem],
        out_specs=pl.BlockSpec((cout, h, w, nl), lambda i: (0, 0, 0, i)),
        scratch_shapes=[pltpu.VMEM((cin, h + 2, w + 2, nl), jnp.float32)],
        compiler_params=cparams,
    )(xt, dw, pmat_s, shift)

    return jnp.transpose(yt, (3, 0, 1, 2))
